# Initial kernel scaffold; baseline (speedup 1.0000x reference)
import jax
import jax.numpy as jnp
from jax.experimental import pallas as pl


def kernel(x, S, w, b, thresh_t):
    raise NotImplementedError("write your pallas kernel here")


# speedup vs baseline: 139.6674x; 139.6674x over previous
import functools

import jax
import jax.numpy as jnp
from jax import lax
from jax.experimental import pallas as pl
from jax.experimental.pallas import tpu as pltpu
from jax.experimental.pallas import tpu_sc as plsc

B = 16384
L = 200
NC = 2
NS = 16
NW = NC * NS
ROWS_PER_W = B // NW
CHUNK = 64
NCHUNK = ROWS_PER_W // CHUNK
GROUPS = CHUNK // 16
UNROLL = 8


def _sc_body(x_hbm, s_hbm, wbt_hbm, logit_hbm, label_hbm,
             idx_v, vals_v, out_logit, out_label, wbt_v, sem):
    c = lax.axis_index("c")
    s = lax.axis_index("s")
    wid = s * NC + c
    row0 = wid * ROWS_PER_W

    pltpu.sync_copy(wbt_hbm, wbt_v)
    w = wbt_v[pl.ds(0, 16)]
    bias = wbt_v[pl.ds(16, 16)]
    thresh = wbt_v[pl.ds(32, 16)]
    lanes = lax.iota(jnp.int32, 16)

    for ci in range(NCHUNK):
        base = (row0 + ci * CHUNK) * L
        pltpu.sync_copy(x_hbm.at[pl.ds(base, CHUNK * L)], idx_v)
        pltpu.async_copy(s_hbm.at[idx_v], vals_v, sem).wait()

        def group_body(g, _):
            base_i = (g * 16 + lanes) * L

            def l_body(j, accs):
                a0, a1 = accs
                l0 = j * UNROLL
                for u in range(0, UNROLL, 2):
                    a0 = a0 + plsc.load_gather(vals_v, [base_i + (l0 + u)])
                    a1 = a1 + plsc.load_gather(vals_v, [base_i + (l0 + u + 1)])
                return (a0, a1)

            zero = jnp.zeros((16,), jnp.float32)
            a0, a1 = lax.fori_loop(0, L // UNROLL, l_body, (zero, zero))
            total = a0 + a1
            logit = total * w + bias
            lab = jnp.where(logit >= thresh, 1.0, 0.0).astype(jnp.float32)
            off = ci * CHUNK + g * 16
            out_logit[pl.ds(off, 16)] = logit
            out_label[pl.ds(off, 16)] = lab
            return 0

        lax.fori_loop(0, GROUPS, group_body, 0)

    pltpu.sync_copy(out_logit, logit_hbm.at[pl.ds(row0, ROWS_PER_W)])
    pltpu.sync_copy(out_label, label_hbm.at[pl.ds(row0, ROWS_PER_W)])


@jax.jit
def _run(x_flat, s_flat, wbt):
    mesh = plsc.VectorSubcoreMesh(core_axis_name="c", subcore_axis_name="s")
    f = functools.partial(
        pl.kernel,
        out_type=[
            jax.ShapeDtypeStruct((B,), jnp.float32),
            jax.ShapeDtypeStruct((B,), jnp.float32),
        ],
        mesh=mesh,
        scratch_types=[
            pltpu.VMEM((CHUNK * L,), jnp.int32),
            pltpu.VMEM((CHUNK * L,), jnp.float32),
            pltpu.VMEM((ROWS_PER_W,), jnp.float32),
            pltpu.VMEM((ROWS_PER_W,), jnp.float32),
            pltpu.VMEM((48,), jnp.float32),
            pltpu.SemaphoreType.DMA,
        ],
        compiler_params=pltpu.CompilerParams(needs_layout_passes=False),
    )(_sc_body)
    return f(x_flat, s_flat, wbt)


def kernel(x, S, w, b, thresh_t):
    x_flat = x.reshape(-1)
    s_flat = S.reshape(-1)
    wbt = jnp.concatenate([
        jnp.full((16,), w[0], jnp.float32),
        jnp.full((16,), b[0], jnp.float32),
        jnp.full((16,), thresh_t[0, 0], jnp.float32),
    ])
    logit_f, label_f = _run(x_flat, s_flat, wbt)
    logit = logit_f.reshape(B, 1)
    label = label_f.reshape(B, 1) != 0.0
    return (logit, label)
